# baseline (speedup 1.0000x reference)
import functools

import jax
import jax.numpy as jnp
from jax import lax
from jax.experimental import pallas as pl
from jax.experimental.pallas import tpu as pltpu
from jax.experimental.pallas import tpu_sc as plsc

_N_CLS = 1000
_N_ENV = 128
_BATCH = 16384

_info = plsc.get_sparse_core_info()
_NC, _NS = _info.num_cores, _info.num_subcores
_NW = _NC * _NS
_BPW = _BATCH // _NW

_RPT = 64
_LAST_ROW0 = _N_CLS - _RPT

_CHUNKS = (64, 128, 160, 160)
_OFFS = (0, 64, 192, 352)


def _softmax_body(x_ref, o_ref):
    x = x_ref[...]
    m = jnp.max(x, axis=-1, keepdims=True)
    e = jnp.exp(x - m)
    o_ref[...] = e * (1.0 / jnp.sum(e, axis=-1, keepdims=True))


def _softmax_table(mat):
    return pl.pallas_call(
        _softmax_body,
        out_shape=jax.ShapeDtypeStruct(mat.shape, mat.dtype),
    )(mat)


_mesh = plsc.VectorSubcoreMesh(core_axis_name="c", subcore_axis_name="s")


@functools.partial(
    pl.kernel,
    mesh=_mesh,
    out_type=jax.ShapeDtypeStruct((_BATCH, _N_ENV), jnp.float32),
    scratch_types=[
        pltpu.VMEM((_BPW,), jnp.int32),
        pltpu.VMEM((_BPW, _N_ENV), jnp.float32),
        pltpu.VMEM_SHARED((_N_CLS, _N_ENV), jnp.float32),
        pltpu.SemaphoreType.DMA,
        pltpu.SemaphoreType.DMA,
        pltpu.SemaphoreType.DMA,
    ],
)
def _gather_sc(table_hbm, idx_hbm, out_hbm, idx_v, buf, shared, isem, gsem,
               wsem):
    c = lax.axis_index("c")
    s = lax.axis_index("s")
    wid = s * _NC + c
    base = wid * _BPW

    idx_cp = pltpu.async_copy(idx_hbm.at[pl.ds(base, _BPW)], idx_v, isem)

    row0 = jnp.minimum(s * _RPT, _LAST_ROW0)
    pltpu.sync_copy(table_hbm.at[pl.ds(row0, _RPT)],
                    shared.at[pl.ds(row0, _RPT)])
    plsc.subcore_barrier()
    idx_cp.wait()

    gathers = [
        pltpu.async_copy(shared.at[idx_v.at[pl.ds(o, n)]],
                         buf.at[pl.ds(o, n)], gsem)
        for o, n in zip(_OFFS, _CHUNKS)
    ]
    writes = []
    for g, o, n in zip(gathers, _OFFS, _CHUNKS):
        g.wait()
        writes.append(
            pltpu.async_copy(buf.at[pl.ds(o, n)],
                             out_hbm.at[pl.ds(base + o, n)], wsem))
    for w in writes:
        w.wait()


def kernel(label, partition_matrix):
    sm = _softmax_table(partition_matrix)
    return _gather_sc(sm, label.astype(jnp.int32))


# speedup vs baseline: 1.0091x; 1.0091x over previous
import functools

import jax
import jax.numpy as jnp
from jax import lax
from jax.experimental import pallas as pl
from jax.experimental.pallas import tpu as pltpu
from jax.experimental.pallas import tpu_sc as plsc

_N_CLS = 1000
_N_ENV = 128
_BATCH = 16384

_info = plsc.get_sparse_core_info()
_NC, _NS = _info.num_cores, _info.num_subcores
_NW = _NC * _NS
_BPW = _BATCH // _NW

_RPT = 64
_LAST_ROW0 = _N_CLS - _RPT

_CH = 128
_NCHUNK = _BPW // _CH


def _softmax_body(x_ref, o_ref):
    x = x_ref[...]
    m = jnp.max(x, axis=-1, keepdims=True)
    e = jnp.exp(x - m)
    o_ref[...] = e * (1.0 / jnp.sum(e, axis=-1, keepdims=True))


def _softmax_table(mat):
    return pl.pallas_call(
        _softmax_body,
        out_shape=jax.ShapeDtypeStruct(mat.shape, mat.dtype),
    )(mat)


_mesh = plsc.VectorSubcoreMesh(core_axis_name="c", subcore_axis_name="s")


@functools.partial(
    pl.kernel,
    mesh=_mesh,
    out_type=jax.ShapeDtypeStruct((_BATCH, _N_ENV), jnp.float32),
    scratch_types=[
        pltpu.VMEM((_BPW,), jnp.int32),
        pltpu.VMEM((_NCHUNK, _CH, _N_ENV), jnp.float32),
        pltpu.VMEM_SHARED((_N_CLS, _N_ENV), jnp.float32),
        pltpu.SemaphoreType.DMA,
        pltpu.SemaphoreType.DMA,
        pltpu.SemaphoreType.DMA,
    ],
)
def _gather_sc(table_hbm, idx_hbm, out_hbm, idx_v, buf, shared, isem, gsem,
               wsem):
    c = lax.axis_index("c")
    s = lax.axis_index("s")
    wid = s * _NC + c
    base = wid * _BPW

    idx_cp = pltpu.async_copy(idx_hbm.at[pl.ds(base, _BPW)], idx_v, isem)

    row0 = jnp.minimum(s * _RPT, _LAST_ROW0)
    pltpu.sync_copy(table_hbm.at[pl.ds(row0, _RPT)],
                    shared.at[pl.ds(row0, _RPT)])
    plsc.subcore_barrier()
    idx_cp.wait()

    gathers = [
        pltpu.async_copy(shared.at[idx_v.at[pl.ds(k * _CH, _CH)]], buf.at[k],
                         gsem)
        for k in range(_NCHUNK)
    ]
    writes = []
    for k in range(_NCHUNK):
        gathers[k].wait()
        writes.append(
            pltpu.async_copy(buf.at[k], out_hbm.at[pl.ds(base + k * _CH, _CH)],
                             wsem))
    for w in writes:
        w.wait()


def kernel(label, partition_matrix):
    sm = _softmax_table(partition_matrix)
    return _gather_sc(sm, label.astype(jnp.int32))
